# Initial kernel scaffold; baseline (speedup 1.0000x reference)
import jax
import jax.numpy as jnp
from jax.experimental import pallas as pl


def kernel(pos, pos_dst, edge_index, basis):
    raise NotImplementedError("write your pallas kernel here")


# speedup vs baseline: 1.7667x; 1.7667x over previous
import functools

import jax
import jax.numpy as jnp
from jax import lax
from jax.experimental import pallas as pl
from jax.experimental.pallas import tpu as pltpu
from jax.experimental.pallas import tpu_sc as plsc

_NC = 2
_NS = 16
_NW = _NC * _NS

_E = 800000
_N = 50000
_ROWS_PER_W = 1563
_ACC_W = 68
_ACC_WORDS = _ROWS_PER_W * _ACC_W
_OUT_STRIDE = 106288
_ACC_ALLOC = 106288
_CH_A = 1000
_PW = 8
_CH_C = 4096
_NCH_C = 196
_COLP = _NCH_C * _CH_C
_FW = 80
_BATCH = 96
_WL_CAP = 5888
_WL_ALLOC = 5952
_DEBUG_XLA_SCATTER = False
_DEBUG_XLA_GATHER = False


def _sc_gather_rows(pos4, posd4, row, col):
    per_w = _E // _NW
    n_ch = per_w // _CH_A
    mesh = plsc.VectorSubcoreMesh(core_axis_name="c", subcore_axis_name="s")

    @functools.partial(
        pl.kernel,
        out_type=(jax.ShapeDtypeStruct((_E, _PW), jnp.float32),
                  jax.ShapeDtypeStruct((_E, _PW), jnp.float32)),
        mesh=mesh,
        scratch_types=[
            pltpu.VMEM((_CH_A,), jnp.int32),
            pltpu.VMEM((_CH_A,), jnp.int32),
            pltpu.VMEM((_CH_A, _PW), jnp.float32),
            pltpu.VMEM((_CH_A, _PW), jnp.float32),
            pltpu.SemaphoreType.DMA,
            pltpu.SemaphoreType.DMA,
        ],
        compiler_params=pltpu.CompilerParams(use_tc_tiling_on_sc=False),
    )
    def k(pos_hbm, posd_hbm, row_hbm, col_hbm, outs_hbm, outd_hbm,
          ridx, cidx, srows, drows, sem1, sem2):
        wid = lax.axis_index("s") * _NC + lax.axis_index("c")

        def chunk(i, carry):
            base = wid * per_w + i * _CH_A
            pltpu.sync_copy(row_hbm.at[pl.ds(base, _CH_A)], ridx)
            pltpu.sync_copy(col_hbm.at[pl.ds(base, _CH_A)], cidx)
            a = pltpu.async_copy(pos_hbm.at[ridx], srows, sem1)
            b = pltpu.async_copy(posd_hbm.at[cidx], drows, sem2)
            a.wait()
            b.wait()
            pltpu.sync_copy(srows, outs_hbm.at[pl.ds(base, _CH_A)])
            pltpu.sync_copy(drows, outd_hbm.at[pl.ds(base, _CH_A)])
            return carry

        lax.fori_loop(0, n_ch, chunk, 0)

    return k(pos4, posd4, row, col)


def _tc_features(src4, dst4, basis4):
    BE = 3200

    def body(s_ref, d_ref, b_ref, o_ref):
        d = s_ref[...] - d_ref[...]
        emb = jnp.dot(d, b_ref[...], preferred_element_type=jnp.float32)
        k = jnp.round(emb * 0.15915494309189535)
        r = emb - k * 6.28125
        r = r - k * 1.9353071795864769e-3
        h = 0.5 * r
        h2 = h * h
        s = h * (1.0 + h2 * (-0.16666667 + h2 *
                             (8.3333333e-3 + h2 * -1.9841270e-4)))
        c = 1.0 + h2 * (-0.5 + h2 * (4.1666667e-2 + h2 *
                                     (-1.3888889e-3 + h2 * 2.4801587e-5)))
        sin_e = 2.0 * s * c
        cos_e = 1.0 - 2.0 * s * s
        o_ref[...] = jnp.concatenate(
            [d[:, 0:4], sin_e, cos_e,
             jnp.zeros((BE, _FW - 68), jnp.float32)], axis=1)

    return pl.pallas_call(
        body,
        grid=(_E // BE,),
        in_specs=[
            pl.BlockSpec((BE, _PW), lambda i: (i, 0)),
            pl.BlockSpec((BE, _PW), lambda i: (i, 0)),
            pl.BlockSpec((_PW, 32), lambda i: (0, 0)),
        ],
        out_specs=pl.BlockSpec((BE, _FW), lambda i: (i, 0)),
        out_shape=jax.ShapeDtypeStruct((_E, _FW), jnp.float32),
    )(src4, dst4, basis4)


def _sc_scatter_max(feat, colp):
    mesh = plsc.VectorSubcoreMesh(core_axis_name="c", subcore_axis_name="s")

    @functools.partial(
        pl.kernel,
        out_type=jax.ShapeDtypeStruct((_NW * _OUT_STRIDE,), jnp.float32),
        mesh=mesh,
        scratch_types=[
            pltpu.VMEM((_CH_C + 16,), jnp.int32),
            pltpu.VMEM((_CH_C + 16,), jnp.int32),
            pltpu.VMEM((_WL_ALLOC,), jnp.int32),
            pltpu.VMEM((_BATCH, _FW), jnp.float32),
            pltpu.VMEM((_ACC_ALLOC,), jnp.float32),
            pltpu.SemaphoreType.DMA,
            pltpu.SemaphoreType.DMA,
            pltpu.SemaphoreType.DMA,
        ],
        compiler_params=pltpu.CompilerParams(
            use_tc_tiling_on_sc=False, needs_layout_passes=False),
    )
    def k(feat_hbm, col_hbm, out_hbm, cb0, cb1, wl, stg, acc, semc0, semc1,
          semg):
        wid = lax.axis_index("s") * _NC + lax.axis_index("c")
        lo = wid * _ROWS_PER_W
        iota16 = lax.iota(jnp.int32, 16)
        neg = jnp.full((16,), -3.4e38, jnp.float32)

        def init(i, c):
            acc[pl.ds(i * 16, 16)] = neg
            return c
        lax.fori_loop(0, _ACC_ALLOC // 16, init, 0)

        def initw(i, c):
            wl[pl.ds(i * 16, 16)] = jnp.zeros((16,), jnp.int32)
            return c
        lax.fori_loop(0, _WL_ALLOC // 16, initw, 0)

        def process(n):
            def batch(b, c2):
                wb = b * _BATCH
                ds = []
                for t in range(_BATCH // 16):
                    wlv = wl[pl.ds(wb + t * 16, 16)]
                    idv = lax.shift_right_logical(wlv, 11)
                    ds.append(pltpu.async_copy(
                        feat_hbm.at[idv], stg.at[pl.ds(t * 16, 16)], semg))
                for d in ds:
                    d.wait()
                ne = jnp.minimum(_BATCH, n - wb)

                def edge(e, c3):
                    packed = wl[pl.ds(wb + e, 16)][0]
                    base = (packed & 2047) * _ACC_W
                    for off5 in (0, 16, 32, 48, 52):
                        a = acc[pl.ds(base + off5, 16)]
                        f = stg[e, pl.ds(off5, 16)]
                        acc[pl.ds(base + off5, 16)] = jnp.maximum(a, f)
                    return c3

                lax.fori_loop(0, ne, edge, 0)
                return c2

            lax.fori_loop(0, (n + _BATCH - 1) // _BATCH, batch, 0)

        def scan_chunk(c, colbuf, off):
            off = lax.cond(off > _WL_CAP - _CH_C,
                           lambda o: (process(o), jnp.int32(0))[1],
                           lambda o: o, off)
            cb = c * _CH_C

            iota_hi = iota16 + 16

            def scan(j, off):
                for u in range(4):
                    jj = j * 4 + u
                    v = colbuf[pl.ds(jj * 16, 16)]
                    lcl = v - lo
                    m = plsc.bitcast(lcl, jnp.uint32) < jnp.uint32(_ROWS_PER_W)
                    cnt = plsc.all_reduce_population_count(m)[0]
                    packed = ((cb + jj * 16 + iota16) * 2048) | lcl
                    key = jnp.where(m, iota16, iota_hi)
                    _, pv = plsc.sort_key_val(key, packed)
                    wl[pl.ds(off, 16)] = pv
                    off = off + cnt
                return off

            return lax.fori_loop(0, _CH_C // 64, scan, off)

        def issue_col(c, buf, sem):
            return pltpu.async_copy(col_hbm.at[pl.ds(c * _CH_C, _CH_C)],
                                    buf.at[pl.ds(0, _CH_C)], sem)

        def wait_col(c, buf, sem):
            pltpu.make_async_copy(col_hbm.at[pl.ds(c * _CH_C, _CH_C)],
                                  buf.at[pl.ds(0, _CH_C)], sem).wait()

        issue_col(0, cb0, semc0)
        issue_col(1, cb1, semc1)

        def pair(i, off):
            c0 = i * 2
            wait_col(c0, cb0, semc0)
            off = scan_chunk(c0, cb0, off)
            issue_col(c0 + 2, cb0, semc0)
            wait_col(c0 + 1, cb1, semc1)
            off = scan_chunk(c0 + 1, cb1, off)
            issue_col(c0 + 3, cb1, semc1)
            return off

        off = lax.fori_loop(0, _NCH_C // 2 - 1, pair, jnp.int32(0))
        wait_col(_NCH_C - 2, cb0, semc0)
        off = scan_chunk(_NCH_C - 2, cb0, off)
        wait_col(_NCH_C - 1, cb1, semc1)
        off = scan_chunk(_NCH_C - 1, cb1, off)
        process(off)

        def fin(i, c):
            x = acc[pl.ds(i * 16, 16)]
            acc[pl.ds(i * 16, 16)] = jnp.where(x < -1e37, 0.0, x)
            return c
        lax.fori_loop(0, _ACC_ALLOC // 16, fin, 0)

        pltpu.sync_copy(acc.at[pl.ds(0, _OUT_STRIDE)],
                        out_hbm.at[pl.ds(wid * _OUT_STRIDE, _OUT_STRIDE)])

    return k(feat, colp)


def kernel(pos, pos_dst, edge_index, basis):
    row = edge_index[0]
    col = edge_index[1]
    pos4 = jnp.pad(pos, ((0, 0), (0, _PW - 3)))
    posd4 = jnp.pad(pos_dst, ((0, 0), (0, _PW - 3)))
    basis4 = jnp.pad(basis, ((0, _PW - 3), (0, 0)))

    src4, dst4 = _sc_gather_rows(pos4, posd4, row, col)
    feat = _tc_features(src4, dst4, basis4)

    colp = jnp.pad(col, (0, _COLP - _E), constant_values=jnp.int32(2**31 - 1))
    out_flat = _sc_scatter_max(feat, colp)
    out68 = (out_flat.reshape(_NW, _OUT_STRIDE)[:, :_ACC_WORDS]
             .reshape(_NW * _ROWS_PER_W, _ACC_W)[:_N])
    return jnp.concatenate([out68[:, 0:3], out68[:, 4:68]], axis=1)


# speedup vs baseline: 1.8809x; 1.0647x over previous
import functools

import jax
import jax.numpy as jnp
from jax import lax
from jax.experimental import pallas as pl
from jax.experimental.pallas import tpu as pltpu
from jax.experimental.pallas import tpu_sc as plsc

_NC = 2
_NS = 16
_NW = _NC * _NS

_E = 800000
_N = 50000
_ROWS_PER_W = 1563
_ACC_W = 68
_ACC_WORDS = _ROWS_PER_W * _ACC_W
_OUT_STRIDE = 106288
_ACC_ALLOC = 106288
_CH_A = 1000
_PW = 8
_CH_C = 4096
_NCH_C = 196
_COLP = _NCH_C * _CH_C
_FW = 80
_BATCH = 48
_WL_CAP = 5888
_WL_ALLOC = 5952
_DEBUG_XLA_SCATTER = False
_DEBUG_XLA_GATHER = False


def _sc_gather_rows(pos4, posd4, row, col):
    per_w = _E // _NW
    n_ch = per_w // _CH_A
    mesh = plsc.VectorSubcoreMesh(core_axis_name="c", subcore_axis_name="s")

    @functools.partial(
        pl.kernel,
        out_type=(jax.ShapeDtypeStruct((_E, _PW), jnp.float32),
                  jax.ShapeDtypeStruct((_E, _PW), jnp.float32)),
        mesh=mesh,
        scratch_types=[
            pltpu.VMEM((_CH_A,), jnp.int32),
            pltpu.VMEM((_CH_A,), jnp.int32),
            pltpu.VMEM((_CH_A, _PW), jnp.float32),
            pltpu.VMEM((_CH_A, _PW), jnp.float32),
            pltpu.SemaphoreType.DMA,
            pltpu.SemaphoreType.DMA,
        ],
        compiler_params=pltpu.CompilerParams(use_tc_tiling_on_sc=False),
    )
    def k(pos_hbm, posd_hbm, row_hbm, col_hbm, outs_hbm, outd_hbm,
          ridx, cidx, srows, drows, sem1, sem2):
        wid = lax.axis_index("s") * _NC + lax.axis_index("c")

        def chunk(i, carry):
            base = wid * per_w + i * _CH_A
            pltpu.sync_copy(row_hbm.at[pl.ds(base, _CH_A)], ridx)
            pltpu.sync_copy(col_hbm.at[pl.ds(base, _CH_A)], cidx)
            a = pltpu.async_copy(pos_hbm.at[ridx], srows, sem1)
            b = pltpu.async_copy(posd_hbm.at[cidx], drows, sem2)
            a.wait()
            b.wait()
            pltpu.sync_copy(srows, outs_hbm.at[pl.ds(base, _CH_A)])
            pltpu.sync_copy(drows, outd_hbm.at[pl.ds(base, _CH_A)])
            return carry

        lax.fori_loop(0, n_ch, chunk, 0)

    return k(pos4, posd4, row, col)


def _tc_features(src4, dst4, basis4):
    BE = 3200

    def body(s_ref, d_ref, b_ref, o_ref):
        d = s_ref[...] - d_ref[...]
        emb = jnp.dot(d, b_ref[...], preferred_element_type=jnp.float32)
        k = jnp.round(emb * 0.15915494309189535)
        r = emb - k * 6.28125
        r = r - k * 1.9353071795864769e-3
        h = 0.5 * r
        h2 = h * h
        s = h * (1.0 + h2 * (-0.16666667 + h2 *
                             (8.3333333e-3 + h2 * -1.9841270e-4)))
        c = 1.0 + h2 * (-0.5 + h2 * (4.1666667e-2 + h2 *
                                     (-1.3888889e-3 + h2 * 2.4801587e-5)))
        sin_e = 2.0 * s * c
        cos_e = 1.0 - 2.0 * s * s
        o_ref[...] = jnp.concatenate(
            [d[:, 0:4], sin_e, cos_e,
             jnp.zeros((BE, _FW - 68), jnp.float32)], axis=1)

    return pl.pallas_call(
        body,
        grid=(_E // BE,),
        in_specs=[
            pl.BlockSpec((BE, _PW), lambda i: (i, 0)),
            pl.BlockSpec((BE, _PW), lambda i: (i, 0)),
            pl.BlockSpec((_PW, 32), lambda i: (0, 0)),
        ],
        out_specs=pl.BlockSpec((BE, _FW), lambda i: (i, 0)),
        out_shape=jax.ShapeDtypeStruct((_E, _FW), jnp.float32),
    )(src4, dst4, basis4)


def _sc_scatter_max(feat, colp):
    mesh = plsc.VectorSubcoreMesh(core_axis_name="c", subcore_axis_name="s")

    @functools.partial(
        pl.kernel,
        out_type=jax.ShapeDtypeStruct((_NW * _OUT_STRIDE,), jnp.float32),
        mesh=mesh,
        scratch_types=[
            pltpu.VMEM((_CH_C + 16,), jnp.int32),
            pltpu.VMEM((_CH_C + 16,), jnp.int32),
            pltpu.VMEM((_WL_ALLOC,), jnp.int32),
            pltpu.VMEM((_BATCH, _FW), jnp.float32),
            pltpu.VMEM((_BATCH, _FW), jnp.float32),
            pltpu.VMEM((_ACC_ALLOC,), jnp.float32),
            pltpu.SemaphoreType.DMA,
            pltpu.SemaphoreType.DMA,
            pltpu.SemaphoreType.DMA,
            pltpu.SemaphoreType.DMA,
        ],
        compiler_params=pltpu.CompilerParams(
            use_tc_tiling_on_sc=False, needs_layout_passes=False),
    )
    def k(feat_hbm, col_hbm, out_hbm, cb0, cb1, wl, stgA, stgB, acc,
          semc0, semc1, semgA, semgB):
        wid = lax.axis_index("s") * _NC + lax.axis_index("c")
        lo = wid * _ROWS_PER_W
        iota16 = lax.iota(jnp.int32, 16)
        neg = jnp.full((16,), -3.4e38, jnp.float32)

        def init(i, c):
            acc[pl.ds(i * 16, 16)] = neg
            return c
        lax.fori_loop(0, _ACC_ALLOC // 16, init, 0)

        def initw(i, c):
            wl[pl.ds(i * 16, 16)] = jnp.zeros((16,), jnp.int32)
            return c
        lax.fori_loop(0, _WL_ALLOC // 16, initw, 0)

        def process(n):
            nb = (n + _BATCH - 1) // _BATCH

            def issueb(b, buf, sem):
                wb = b * _BATCH
                for t in range(_BATCH // 16):
                    wlv = wl[pl.ds(wb + t * 16, 16)]
                    idv = lax.shift_right_logical(wlv, 11)
                    pltpu.async_copy(
                        feat_hbm.at[idv], buf.at[pl.ds(t * 16, 16)], sem)

            def waitb(b, buf, sem):
                wb = b * _BATCH
                for t in range(_BATCH // 16):
                    wlv = wl[pl.ds(wb + t * 16, 16)]
                    idv = lax.shift_right_logical(wlv, 11)
                    pltpu.make_async_copy(
                        feat_hbm.at[idv], buf.at[pl.ds(t * 16, 16)],
                        sem).wait()

            def do_batch(b, buf):
                wb = b * _BATCH
                ne = jnp.minimum(_BATCH, n - wb)

                def edge(e, c3):
                    packed = wl[pl.ds(wb + e, 16)][0]
                    base = (packed & 2047) * _ACC_W
                    for off5 in (0, 16, 32, 48, 52):
                        a = acc[pl.ds(base + off5, 16)]
                        f = buf[e, pl.ds(off5, 16)]
                        acc[pl.ds(base + off5, 16)] = jnp.maximum(a, f)
                    return c3

                lax.fori_loop(0, ne, edge, 0)

            def step(b, c2):
                def even(_):
                    waitb(b, stgA, semgA)
                    lax.cond(b + 1 < nb,
                             lambda __: issueb(b + 1, stgB, semgB),
                             lambda __: None, 0)
                    do_batch(b, stgA)
                    return 0

                def odd(_):
                    waitb(b, stgB, semgB)
                    lax.cond(b + 1 < nb,
                             lambda __: issueb(b + 1, stgA, semgA),
                             lambda __: None, 0)
                    do_batch(b, stgB)
                    return 0

                return lax.cond(b % 2 == 0, even, odd, 0)

            lax.cond(nb > 0, lambda _: (issueb(0, stgA, semgA), 0)[1],
                     lambda _: 0, 0)
            lax.fori_loop(0, nb, step, 0)

        def scan_chunk(c, colbuf, off):
            off = lax.cond(off > _WL_CAP - _CH_C,
                           lambda o: (process(o), jnp.int32(0))[1],
                           lambda o: o, off)
            cb = c * _CH_C

            iota_hi = iota16 + 16

            def scan(j, off):
                for u in range(4):
                    jj = j * 4 + u
                    v = colbuf[pl.ds(jj * 16, 16)]
                    lcl = v - lo
                    m = plsc.bitcast(lcl, jnp.uint32) < jnp.uint32(_ROWS_PER_W)
                    cnt = plsc.all_reduce_population_count(m)[0]
                    packed = ((cb + jj * 16 + iota16) * 2048) | lcl
                    key = jnp.where(m, iota16, iota_hi)
                    _, pv = plsc.sort_key_val(key, packed)
                    wl[pl.ds(off, 16)] = pv
                    off = off + cnt
                return off

            return lax.fori_loop(0, _CH_C // 64, scan, off)

        def issue_col(c, buf, sem):
            return pltpu.async_copy(col_hbm.at[pl.ds(c * _CH_C, _CH_C)],
                                    buf.at[pl.ds(0, _CH_C)], sem)

        def wait_col(c, buf, sem):
            pltpu.make_async_copy(col_hbm.at[pl.ds(c * _CH_C, _CH_C)],
                                  buf.at[pl.ds(0, _CH_C)], sem).wait()

        issue_col(0, cb0, semc0)
        issue_col(1, cb1, semc1)

        def pair(i, off):
            c0 = i * 2
            wait_col(c0, cb0, semc0)
            off = scan_chunk(c0, cb0, off)
            issue_col(c0 + 2, cb0, semc0)
            wait_col(c0 + 1, cb1, semc1)
            off = scan_chunk(c0 + 1, cb1, off)
            issue_col(c0 + 3, cb1, semc1)
            return off

        off = lax.fori_loop(0, _NCH_C // 2 - 1, pair, jnp.int32(0))
        wait_col(_NCH_C - 2, cb0, semc0)
        off = scan_chunk(_NCH_C - 2, cb0, off)
        wait_col(_NCH_C - 1, cb1, semc1)
        off = scan_chunk(_NCH_C - 1, cb1, off)
        process(off)

        def fin(i, c):
            x = acc[pl.ds(i * 16, 16)]
            acc[pl.ds(i * 16, 16)] = jnp.where(x < -1e37, 0.0, x)
            return c
        lax.fori_loop(0, _ACC_ALLOC // 16, fin, 0)

        pltpu.sync_copy(acc.at[pl.ds(0, _OUT_STRIDE)],
                        out_hbm.at[pl.ds(wid * _OUT_STRIDE, _OUT_STRIDE)])

    return k(feat, colp)


def kernel(pos, pos_dst, edge_index, basis):
    row = edge_index[0]
    col = edge_index[1]
    pos4 = jnp.pad(pos, ((0, 0), (0, _PW - 3)))
    posd4 = jnp.pad(pos_dst, ((0, 0), (0, _PW - 3)))
    basis4 = jnp.pad(basis, ((0, _PW - 3), (0, 0)))

    src4, dst4 = _sc_gather_rows(pos4, posd4, row, col)
    feat = _tc_features(src4, dst4, basis4)

    colp = jnp.pad(col, (0, _COLP - _E), constant_values=jnp.int32(2**31 - 1))
    out_flat = _sc_scatter_max(feat, colp)
    out68 = (out_flat.reshape(_NW, _OUT_STRIDE)[:, :_ACC_WORDS]
             .reshape(_NW * _ROWS_PER_W, _ACC_W)[:_N])
    return jnp.concatenate([out68[:, 0:3], out68[:, 4:68]], axis=1)


# speedup vs baseline: 2.1631x; 1.1500x over previous
import functools

import jax
import jax.numpy as jnp
from jax import lax
from jax.experimental import pallas as pl
from jax.experimental.pallas import tpu as pltpu
from jax.experimental.pallas import tpu_sc as plsc

_NC = 2
_NS = 16
_NW = _NC * _NS

_E = 800000
_N = 50000
_ROWS_PER_W = 1563
_ACC_W = 68
_ACC_WORDS = _ROWS_PER_W * _ACC_W
_OUT_STRIDE = 106288
_ACC_ALLOC = 106288
_CH_A = 1000
_PW = 8
_CH_C = 4096
_EH = 400000
_NCH_C = 98
_COLP = _NCH_C * _CH_C
_FW = 80
_BATCH = 48
_WL_CAP = 5888
_WL_ALLOC = 5952
_DEBUG_XLA_SCATTER = False
_DEBUG_XLA_GATHER = False


def _sc_gather_rows(pos4, posd4, row, col):
    per_w = _E // _NW
    n_ch = per_w // _CH_A
    mesh = plsc.VectorSubcoreMesh(core_axis_name="c", subcore_axis_name="s")

    @functools.partial(
        pl.kernel,
        out_type=(jax.ShapeDtypeStruct((_E, _PW), jnp.float32),
                  jax.ShapeDtypeStruct((_E, _PW), jnp.float32)),
        mesh=mesh,
        scratch_types=[
            pltpu.VMEM((_CH_A,), jnp.int32),
            pltpu.VMEM((_CH_A,), jnp.int32),
            pltpu.VMEM((_CH_A, _PW), jnp.float32),
            pltpu.VMEM((_CH_A, _PW), jnp.float32),
            pltpu.SemaphoreType.DMA,
            pltpu.SemaphoreType.DMA,
        ],
        compiler_params=pltpu.CompilerParams(use_tc_tiling_on_sc=False),
    )
    def k(pos_hbm, posd_hbm, row_hbm, col_hbm, outs_hbm, outd_hbm,
          ridx, cidx, srows, drows, sem1, sem2):
        wid = lax.axis_index("s") * _NC + lax.axis_index("c")

        def chunk(i, carry):
            base = wid * per_w + i * _CH_A
            pltpu.sync_copy(row_hbm.at[pl.ds(base, _CH_A)], ridx)
            pltpu.sync_copy(col_hbm.at[pl.ds(base, _CH_A)], cidx)
            a = pltpu.async_copy(pos_hbm.at[ridx], srows, sem1)
            b = pltpu.async_copy(posd_hbm.at[cidx], drows, sem2)
            a.wait()
            b.wait()
            pltpu.sync_copy(srows, outs_hbm.at[pl.ds(base, _CH_A)])
            pltpu.sync_copy(drows, outd_hbm.at[pl.ds(base, _CH_A)])
            return carry

        lax.fori_loop(0, n_ch, chunk, 0)

    return k(pos4, posd4, row, col)


def _tc_features(src4, dst4, basis4, base):
    BE = 3200

    def body(s_ref, d_ref, b_ref, o_ref):
        d = s_ref[...] - d_ref[...]
        emb = jnp.dot(d, b_ref[...], preferred_element_type=jnp.float32)
        k = jnp.round(emb * 0.15915494309189535)
        r = emb - k * 6.28125
        r = r - k * 1.9353071795864769e-3
        h = 0.5 * r
        h2 = h * h
        s = h * (1.0 + h2 * (-0.16666667 + h2 *
                             (8.3333333e-3 + h2 * -1.9841270e-4)))
        c = 1.0 + h2 * (-0.5 + h2 * (4.1666667e-2 + h2 *
                                     (-1.3888889e-3 + h2 * 2.4801587e-5)))
        sin_e = 2.0 * s * c
        cos_e = 1.0 - 2.0 * s * s
        o_ref[...] = jnp.concatenate(
            [d[:, 0:4], sin_e, cos_e,
             jnp.zeros((BE, _FW - 68), jnp.float32)], axis=1)

    return pl.pallas_call(
        body,
        grid=(_EH // BE,),
        in_specs=[
            pl.BlockSpec((BE, _PW), lambda i: (base + i, 0)),
            pl.BlockSpec((BE, _PW), lambda i: (base + i, 0)),
            pl.BlockSpec((_PW, 32), lambda i: (0, 0)),
        ],
        out_specs=pl.BlockSpec((BE, _FW), lambda i: (i, 0)),
        out_shape=jax.ShapeDtypeStruct((_EH, _FW), jnp.float32),
    )(src4, dst4, basis4)


def _sc_scatter_max(feat, colp, prev):
    mesh = plsc.VectorSubcoreMesh(core_axis_name="c", subcore_axis_name="s")

    @functools.partial(
        pl.kernel,
        out_type=jax.ShapeDtypeStruct((_NW * _OUT_STRIDE,), jnp.float32),
        mesh=mesh,
        scratch_types=[
            pltpu.VMEM((_CH_C + 16,), jnp.int32),
            pltpu.VMEM((_CH_C + 16,), jnp.int32),
            pltpu.VMEM((_WL_ALLOC,), jnp.int32),
            pltpu.VMEM((_BATCH, _FW), jnp.float32),
            pltpu.VMEM((_BATCH, _FW), jnp.float32),
            pltpu.VMEM((_ACC_ALLOC,), jnp.float32),
            pltpu.SemaphoreType.DMA,
            pltpu.SemaphoreType.DMA,
            pltpu.SemaphoreType.DMA,
            pltpu.SemaphoreType.DMA,
        ],
        compiler_params=pltpu.CompilerParams(
            use_tc_tiling_on_sc=False, needs_layout_passes=False),
    )
    def k(feat_hbm, col_hbm, *rest):
        if prev is None:
            out_hbm, cb0, cb1, wl, stgA, stgB, acc, \
                semc0, semc1, semgA, semgB = rest
        else:
            prev_hbm, out_hbm, cb0, cb1, wl, stgA, stgB, acc, \
                semc0, semc1, semgA, semgB = rest
        wid = lax.axis_index("s") * _NC + lax.axis_index("c")
        lo = wid * _ROWS_PER_W
        iota16 = lax.iota(jnp.int32, 16)
        neg = jnp.full((16,), -3.4e38, jnp.float32)

        if prev is None:
            def init(i, c):
                acc[pl.ds(i * 16, 16)] = neg
                return c
            lax.fori_loop(0, _ACC_ALLOC // 16, init, 0)
        else:
            pltpu.sync_copy(
                prev_hbm.at[pl.ds(wid * _OUT_STRIDE, _OUT_STRIDE)],
                acc.at[pl.ds(0, _OUT_STRIDE)])

        def initw(i, c):
            wl[pl.ds(i * 16, 16)] = jnp.zeros((16,), jnp.int32)
            return c
        lax.fori_loop(0, _WL_ALLOC // 16, initw, 0)

        def process(n):
            nb = (n + _BATCH - 1) // _BATCH

            def issueb(b, buf, sem):
                wb = b * _BATCH
                for t in range(_BATCH // 16):
                    wlv = wl[pl.ds(wb + t * 16, 16)]
                    idv = lax.shift_right_logical(wlv, 11)
                    pltpu.async_copy(
                        feat_hbm.at[idv], buf.at[pl.ds(t * 16, 16)], sem)

            def waitb(b, buf, sem):
                wb = b * _BATCH
                for t in range(_BATCH // 16):
                    wlv = wl[pl.ds(wb + t * 16, 16)]
                    idv = lax.shift_right_logical(wlv, 11)
                    pltpu.make_async_copy(
                        feat_hbm.at[idv], buf.at[pl.ds(t * 16, 16)],
                        sem).wait()

            def do_batch(b, buf):
                wb = b * _BATCH
                ne = jnp.minimum(_BATCH, n - wb)

                def edge(e, c3):
                    packed = wl[pl.ds(wb + e, 16)][0]
                    base = (packed & 2047) * _ACC_W
                    for off5 in (0, 16, 32, 48, 52):
                        a = acc[pl.ds(base + off5, 16)]
                        f = buf[e, pl.ds(off5, 16)]
                        acc[pl.ds(base + off5, 16)] = jnp.maximum(a, f)
                    return c3

                lax.fori_loop(0, ne, edge, 0)

            def step(b, c2):
                def even(_):
                    waitb(b, stgA, semgA)
                    lax.cond(b + 1 < nb,
                             lambda __: issueb(b + 1, stgB, semgB),
                             lambda __: None, 0)
                    do_batch(b, stgA)
                    return 0

                def odd(_):
                    waitb(b, stgB, semgB)
                    lax.cond(b + 1 < nb,
                             lambda __: issueb(b + 1, stgA, semgA),
                             lambda __: None, 0)
                    do_batch(b, stgB)
                    return 0

                return lax.cond(b % 2 == 0, even, odd, 0)

            lax.cond(nb > 0, lambda _: (issueb(0, stgA, semgA), 0)[1],
                     lambda _: 0, 0)
            lax.fori_loop(0, nb, step, 0)

        def scan_chunk(c, colbuf, off):
            off = lax.cond(off > _WL_CAP - _CH_C,
                           lambda o: (process(o), jnp.int32(0))[1],
                           lambda o: o, off)
            cb = c * _CH_C

            iota_hi = iota16 + 16

            def scan(j, off):
                for u in range(4):
                    jj = j * 4 + u
                    v = colbuf[pl.ds(jj * 16, 16)]
                    lcl = v - lo
                    m = plsc.bitcast(lcl, jnp.uint32) < jnp.uint32(_ROWS_PER_W)
                    cnt = plsc.all_reduce_population_count(m)[0]
                    packed = ((cb + jj * 16 + iota16) * 2048) | lcl
                    key = jnp.where(m, iota16, iota_hi)
                    _, pv = plsc.sort_key_val(key, packed)
                    wl[pl.ds(off, 16)] = pv
                    off = off + cnt
                return off

            return lax.fori_loop(0, _CH_C // 64, scan, off)

        def issue_col(c, buf, sem):
            return pltpu.async_copy(col_hbm.at[pl.ds(c * _CH_C, _CH_C)],
                                    buf.at[pl.ds(0, _CH_C)], sem)

        def wait_col(c, buf, sem):
            pltpu.make_async_copy(col_hbm.at[pl.ds(c * _CH_C, _CH_C)],
                                  buf.at[pl.ds(0, _CH_C)], sem).wait()

        issue_col(0, cb0, semc0)
        issue_col(1, cb1, semc1)

        def pair(i, off):
            c0 = i * 2
            wait_col(c0, cb0, semc0)
            off = scan_chunk(c0, cb0, off)
            issue_col(c0 + 2, cb0, semc0)
            wait_col(c0 + 1, cb1, semc1)
            off = scan_chunk(c0 + 1, cb1, off)
            issue_col(c0 + 3, cb1, semc1)
            return off

        off = lax.fori_loop(0, _NCH_C // 2 - 1, pair, jnp.int32(0))
        wait_col(_NCH_C - 2, cb0, semc0)
        off = scan_chunk(_NCH_C - 2, cb0, off)
        wait_col(_NCH_C - 1, cb1, semc1)
        off = scan_chunk(_NCH_C - 1, cb1, off)
        process(off)

        if prev is not None:
            def fin(i, c):
                x = acc[pl.ds(i * 16, 16)]
                acc[pl.ds(i * 16, 16)] = jnp.where(x < -1e37, 0.0, x)
                return c
            lax.fori_loop(0, _ACC_ALLOC // 16, fin, 0)

        pltpu.sync_copy(acc.at[pl.ds(0, _OUT_STRIDE)],
                        out_hbm.at[pl.ds(wid * _OUT_STRIDE, _OUT_STRIDE)])

    if prev is None:
        return k(feat, colp)
    return k(feat, colp, prev)


def kernel(pos, pos_dst, edge_index, basis):
    row = edge_index[0]
    col = edge_index[1]
    pos4 = jnp.pad(pos, ((0, 0), (0, _PW - 3)))
    posd4 = jnp.pad(pos_dst, ((0, 0), (0, _PW - 3)))
    basis4 = jnp.pad(basis, ((0, _PW - 3), (0, 0)))

    src4, dst4 = _sc_gather_rows(pos4, posd4, row, col)
    feat0 = _tc_features(src4, dst4, basis4, 0)
    feat1 = _tc_features(src4, dst4, basis4, _EH // 3200)

    pad_v = jnp.int32(2**31 - 1)
    col0 = jnp.pad(col[:_EH], (0, _COLP - _EH), constant_values=pad_v)
    col1 = jnp.pad(col[_EH:], (0, _COLP - _EH), constant_values=pad_v)
    part = _sc_scatter_max(feat0, col0, None)
    out_flat = _sc_scatter_max(feat1, col1, part)
    out68 = (out_flat.reshape(_NW, _OUT_STRIDE)[:, :_ACC_WORDS]
             .reshape(_NW * _ROWS_PER_W, _ACC_W)[:_N])
    return jnp.concatenate([out68[:, 0:3], out68[:, 4:68]], axis=1)


# speedup vs baseline: 2.1652x; 1.0010x over previous
import functools

import jax
import jax.numpy as jnp
from jax import lax
from jax.experimental import pallas as pl
from jax.experimental.pallas import tpu as pltpu
from jax.experimental.pallas import tpu_sc as plsc

_NC = 2
_NS = 16
_NW = _NC * _NS

_E = 800000
_N = 50000
_ROWS_PER_W = 1563
_ACC_W = 68
_ACC_WORDS = _ROWS_PER_W * _ACC_W
_OUT_STRIDE = 106288
_ACC_ALLOC = 106288
_CH_A = 1000
_PW = 8
_CH_C = 4096
_EH = 400000
_NCH_C = 98
_COLP = _NCH_C * _CH_C
_FW = 80
_BATCH = 48
_WL_CAP = 5888
_WL_ALLOC = 5952


def _sc_gather_rows(pos4, posd4, row, col):
    per_w = _E // _NW
    n_ch = per_w // _CH_A
    mesh = plsc.VectorSubcoreMesh(core_axis_name="c", subcore_axis_name="s")

    @functools.partial(
        pl.kernel,
        out_type=(jax.ShapeDtypeStruct((_E, _PW), jnp.float32),
                  jax.ShapeDtypeStruct((_E, _PW), jnp.float32)),
        mesh=mesh,
        scratch_types=[
            pltpu.VMEM((_CH_A,), jnp.int32),
            pltpu.VMEM((_CH_A,), jnp.int32),
            pltpu.VMEM((_CH_A, _PW), jnp.float32),
            pltpu.VMEM((_CH_A, _PW), jnp.float32),
            pltpu.SemaphoreType.DMA,
            pltpu.SemaphoreType.DMA,
        ],
        compiler_params=pltpu.CompilerParams(use_tc_tiling_on_sc=False),
    )
    def k(pos_hbm, posd_hbm, row_hbm, col_hbm, outs_hbm, outd_hbm,
          ridx, cidx, srows, drows, sem1, sem2):
        wid = lax.axis_index("s") * _NC + lax.axis_index("c")

        def chunk(i, carry):
            base = wid * per_w + i * _CH_A
            pltpu.sync_copy(row_hbm.at[pl.ds(base, _CH_A)], ridx)
            pltpu.sync_copy(col_hbm.at[pl.ds(base, _CH_A)], cidx)
            a = pltpu.async_copy(pos_hbm.at[ridx], srows, sem1)
            b = pltpu.async_copy(posd_hbm.at[cidx], drows, sem2)
            a.wait()
            b.wait()
            pltpu.sync_copy(srows, outs_hbm.at[pl.ds(base, _CH_A)])
            pltpu.sync_copy(drows, outd_hbm.at[pl.ds(base, _CH_A)])
            return carry

        lax.fori_loop(0, n_ch, chunk, 0)

    return k(pos4, posd4, row, col)


def _tc_features(src4, dst4, basis4, base):
    BE = 3200

    def body(s_ref, d_ref, b_ref, o_ref):
        d = s_ref[...] - d_ref[...]
        emb = jnp.dot(d, b_ref[...], preferred_element_type=jnp.float32)
        k = jnp.round(emb * 0.15915494309189535)
        r = emb - k * 6.28125
        r = r - k * 1.9353071795864769e-3
        h = 0.5 * r
        h2 = h * h
        s = h * (1.0 + h2 * (-0.16666667 + h2 *
                             (8.3333333e-3 + h2 * -1.9841270e-4)))
        c = 1.0 + h2 * (-0.5 + h2 * (4.1666667e-2 + h2 *
                                     (-1.3888889e-3 + h2 * 2.4801587e-5)))
        sin_e = 2.0 * s * c
        cos_e = 1.0 - 2.0 * s * s
        o_ref[...] = jnp.concatenate(
            [d[:, 0:4], sin_e, cos_e,
             jnp.zeros((BE, _FW - 68), jnp.float32)], axis=1)

    return pl.pallas_call(
        body,
        grid=(_EH // BE,),
        in_specs=[
            pl.BlockSpec((BE, _PW), lambda i: (base + i, 0)),
            pl.BlockSpec((BE, _PW), lambda i: (base + i, 0)),
            pl.BlockSpec((_PW, 32), lambda i: (0, 0)),
        ],
        out_specs=pl.BlockSpec((BE, _FW), lambda i: (i, 0)),
        out_shape=jax.ShapeDtypeStruct((_EH, _FW), jnp.float32),
    )(src4, dst4, basis4)


def _sc_scatter_max(feat, colp, prev):
    mesh = plsc.VectorSubcoreMesh(core_axis_name="c", subcore_axis_name="s")

    @functools.partial(
        pl.kernel,
        out_type=jax.ShapeDtypeStruct((_NW * _OUT_STRIDE,), jnp.float32),
        mesh=mesh,
        scratch_types=[
            pltpu.VMEM((_CH_C + 16,), jnp.int32),
            pltpu.VMEM((_CH_C + 16,), jnp.int32),
            pltpu.VMEM((_WL_ALLOC,), jnp.int32),
            pltpu.VMEM((_BATCH, _FW), jnp.float32),
            pltpu.VMEM((_BATCH, _FW), jnp.float32),
            pltpu.VMEM((_ACC_ALLOC,), jnp.float32),
            pltpu.SemaphoreType.DMA,
            pltpu.SemaphoreType.DMA,
            pltpu.SemaphoreType.DMA,
            pltpu.SemaphoreType.DMA,
        ],
        compiler_params=pltpu.CompilerParams(
            use_tc_tiling_on_sc=False, needs_layout_passes=False),
    )
    def k(feat_hbm, col_hbm, *rest):
        if prev is None:
            out_hbm, cb0, cb1, wl, stgA, stgB, acc, \
                semc0, semc1, semgA, semgB = rest
        else:
            prev_hbm, out_hbm, cb0, cb1, wl, stgA, stgB, acc, \
                semc0, semc1, semgA, semgB = rest
        wid = lax.axis_index("s") * _NC + lax.axis_index("c")
        lo = wid * _ROWS_PER_W
        iota16 = lax.iota(jnp.int32, 16)
        neg = jnp.full((16,), -3.4e38, jnp.float32)

        if prev is None:
            def init(i, c):
                acc[pl.ds(i * 16, 16)] = neg
                return c
            lax.fori_loop(0, _ACC_ALLOC // 16, init, 0)
        else:
            pltpu.sync_copy(
                prev_hbm.at[pl.ds(wid * _OUT_STRIDE, _OUT_STRIDE)],
                acc.at[pl.ds(0, _OUT_STRIDE)])

        def initw(i, c):
            wl[pl.ds(i * 16, 16)] = jnp.zeros((16,), jnp.int32)
            return c
        lax.fori_loop(0, _WL_ALLOC // 16, initw, 0)

        def process(n):
            nb = (n + _BATCH - 1) // _BATCH

            def issueb(b, buf, sem):
                wb = b * _BATCH
                for t in range(_BATCH // 16):
                    wlv = wl[pl.ds(wb + t * 16, 16)]
                    idv = lax.shift_right_logical(wlv, 11)
                    pltpu.async_copy(
                        feat_hbm.at[idv], buf.at[pl.ds(t * 16, 16)], sem)

            def waitb(b, buf, sem):
                wb = b * _BATCH
                for t in range(_BATCH // 16):
                    wlv = wl[pl.ds(wb + t * 16, 16)]
                    idv = lax.shift_right_logical(wlv, 11)
                    pltpu.make_async_copy(
                        feat_hbm.at[idv], buf.at[pl.ds(t * 16, 16)],
                        sem).wait()

            def do_batch(b, buf):
                wb = b * _BATCH
                ne = jnp.minimum(_BATCH, n - wb)

                def edge(e, c3):
                    packed = wl[pl.ds(wb + e, 16)][0]
                    base = (packed & 2047) * _ACC_W
                    for off5 in (0, 16, 32, 48, 52):
                        a = acc[pl.ds(base + off5, 16)]
                        f = buf[e, pl.ds(off5, 16)]
                        acc[pl.ds(base + off5, 16)] = jnp.maximum(a, f)
                    return c3

                lax.fori_loop(0, ne, edge, 0)

            def step(b, c2):
                def even(_):
                    waitb(b, stgA, semgA)
                    lax.cond(b + 1 < nb,
                             lambda __: issueb(b + 1, stgB, semgB),
                             lambda __: None, 0)
                    do_batch(b, stgA)
                    return 0

                def odd(_):
                    waitb(b, stgB, semgB)
                    lax.cond(b + 1 < nb,
                             lambda __: issueb(b + 1, stgA, semgA),
                             lambda __: None, 0)
                    do_batch(b, stgB)
                    return 0

                return lax.cond(b % 2 == 0, even, odd, 0)

            lax.cond(nb > 0, lambda _: (issueb(0, stgA, semgA), 0)[1],
                     lambda _: 0, 0)
            lax.fori_loop(0, nb, step, 0)

        def scan_chunk(c, colbuf, off):
            off = lax.cond(off > _WL_CAP - _CH_C,
                           lambda o: (process(o), jnp.int32(0))[1],
                           lambda o: o, off)
            cb = c * _CH_C

            iota_hi = iota16 + 16

            def scan(j, off):
                for u in range(4):
                    jj = j * 4 + u
                    v = colbuf[pl.ds(jj * 16, 16)]
                    lcl = v - lo
                    m = plsc.bitcast(lcl, jnp.uint32) < jnp.uint32(_ROWS_PER_W)
                    cnt = plsc.all_reduce_population_count(m)[0]
                    packed = ((cb + jj * 16 + iota16) * 2048) | lcl
                    key = jnp.where(m, iota16, iota_hi)
                    _, pv = plsc.sort_key_val(key, packed)
                    wl[pl.ds(off, 16)] = pv
                    off = off + cnt
                return off

            return lax.fori_loop(0, _CH_C // 64, scan, off)

        def issue_col(c, buf, sem):
            return pltpu.async_copy(col_hbm.at[pl.ds(c * _CH_C, _CH_C)],
                                    buf.at[pl.ds(0, _CH_C)], sem)

        def wait_col(c, buf, sem):
            pltpu.make_async_copy(col_hbm.at[pl.ds(c * _CH_C, _CH_C)],
                                  buf.at[pl.ds(0, _CH_C)], sem).wait()

        issue_col(0, cb0, semc0)
        issue_col(1, cb1, semc1)

        def pair(i, off):
            c0 = i * 2
            wait_col(c0, cb0, semc0)
            off = scan_chunk(c0, cb0, off)
            issue_col(c0 + 2, cb0, semc0)
            wait_col(c0 + 1, cb1, semc1)
            off = scan_chunk(c0 + 1, cb1, off)
            issue_col(c0 + 3, cb1, semc1)
            return off

        off = lax.fori_loop(0, _NCH_C // 2 - 1, pair, jnp.int32(0))
        wait_col(_NCH_C - 2, cb0, semc0)
        off = scan_chunk(_NCH_C - 2, cb0, off)
        wait_col(_NCH_C - 1, cb1, semc1)
        off = scan_chunk(_NCH_C - 1, cb1, off)
        process(off)

        if prev is not None:
            def fin(i, c):
                x = acc[pl.ds(i * 16, 16)]
                acc[pl.ds(i * 16, 16)] = jnp.where(x < -1e37, 0.0, x)
                return c
            lax.fori_loop(0, _ACC_ALLOC // 16, fin, 0)

        pltpu.sync_copy(acc.at[pl.ds(0, _OUT_STRIDE)],
                        out_hbm.at[pl.ds(wid * _OUT_STRIDE, _OUT_STRIDE)])

    if prev is None:
        return k(feat, colp)
    return k(feat, colp, prev)


def kernel(pos, pos_dst, edge_index, basis):
    row = edge_index[0]
    col = edge_index[1]
    pos4 = jnp.pad(pos, ((0, 0), (0, _PW - 3)))
    posd4 = jnp.pad(pos_dst, ((0, 0), (0, _PW - 3)))
    basis4 = jnp.pad(basis, ((0, _PW - 3), (0, 0)))

    src4, dst4 = _sc_gather_rows(pos4, posd4, row, col)
    feat0 = _tc_features(src4, dst4, basis4, 0)
    feat1 = _tc_features(src4, dst4, basis4, _EH // 3200)

    pad_v = jnp.int32(2**31 - 1)
    col0 = jnp.pad(col[:_EH], (0, _COLP - _EH), constant_values=pad_v)
    col1 = jnp.pad(col[_EH:], (0, _COLP - _EH), constant_values=pad_v)
    part = _sc_scatter_max(feat0, col0, None)
    out_flat = _sc_scatter_max(feat1, col1, part)
    out68 = (out_flat.reshape(_NW, _OUT_STRIDE)[:, :_ACC_WORDS]
             .reshape(_NW * _ROWS_PER_W, _ACC_W)[:_N])
    return jnp.concatenate([out68[:, 0:3], out68[:, 4:68]], axis=1)
